# baseline (speedup 1.0000x reference)
import functools

import jax
import jax.numpy as jnp
from jax import lax
from jax.experimental import pallas as pl
from jax.experimental.pallas import tpu as pltpu
from jax.experimental.pallas import tpu_sc as plsc

D = 128
_NC = 2
_NS = 16
_NW = _NC * _NS
_CHUNK = 128
_BN = 8192


def _sc_gather(value_vocab, path_vocab, idx_flat, n):
    per_col = n // _NW
    nch = per_col // _CHUNK
    mesh = plsc.VectorSubcoreMesh(core_axis_name="c", subcore_axis_name="s",
                                  num_cores=_NC)
    nbuf = 4

    @functools.partial(
        pl.kernel,
        mesh=mesh,
        out_type=jax.ShapeDtypeStruct((3 * n, D), jnp.float32),
        scratch_types=(
            [pltpu.VMEM((_CHUNK,), jnp.int32) for _ in range(nbuf)]
            + [pltpu.VMEM((_CHUNK, D), jnp.float32) for _ in range(nbuf)]
            + [pltpu.SemaphoreType.DMA for _ in range(2 * nbuf)]
        ),
    )
    def gather_kernel(vv, pv, idxh, out, *scratch):
        idx_bufs = scratch[0:nbuf]
        row_bufs = scratch[nbuf:2 * nbuf]
        gsems = scratch[2 * nbuf:3 * nbuf]
        wsems = scratch[3 * nbuf:4 * nbuf]
        wid = lax.axis_index("s") * _NC + lax.axis_index("c")
        tables = (vv, pv, vv)
        jobs = []
        for c in range(3):
            base0 = c * n + wid * per_col
            for j in range(nch):
                jobs.append((base0 + j * _CHUNK, tables[c]))
        njobs = len(jobs)
        gcp = [None] * nbuf
        wcp = [None] * nbuf

        def fire_gather(k):
            base, table = jobs[k]
            b = k % nbuf
            pltpu.sync_copy(idxh.at[pl.ds(base, _CHUNK)], idx_bufs[b])
            gcp[b] = pltpu.async_copy(table.at[idx_bufs[b]], row_bufs[b], gsems[b])

        for k in range(min(nbuf, njobs)):
            fire_gather(k)
        for k in range(njobs):
            b = k % nbuf
            gcp[b].wait()
            wcp[b] = pltpu.async_copy(row_bufs[b],
                                      out.at[pl.ds(jobs[k][0], _CHUNK)], wsems[b])
            if k + nbuf < njobs:
                wcp[b].wait()
                fire_gather(k + nbuf)
        for k in range(max(0, njobs - nbuf), njobs):
            wcp[k % nbuf].wait()

    return gather_kernel(value_vocab, path_vocab, idx_flat)


def _attn_body(ctx0, ctx1, ctx2, w_ref, a_ref, wd_ref, b_ref, out_ref,
               m_ref, s_ref, v_ref, *, nblk):
    i = pl.program_id(0)

    @pl.when(i == 0)
    def _():
        m_ref[0, 0] = -1e30
        s_ref[0, 0] = 0.0
        v_ref[...] = jnp.zeros_like(v_ref)

    cw = None
    for c, ref in enumerate((ctx0, ctx1, ctx2)):
        wc = w_ref[:, c * D:(c + 1) * D]
        part = lax.dot_general(ref[0], wc, (((1,), (1,)), ((), ())),
                               preferred_element_type=jnp.float32)
        cw = part if cw is None else cw + part
    combined = jnp.tanh(cw)
    z = lax.dot_general(combined, a_ref[...], (((1,), (1,)), ((), ())),
                        preferred_element_type=jnp.float32)
    m_old = m_ref[0, 0]
    m_new = jnp.maximum(m_old, jnp.max(z))
    corr = jnp.exp(m_old - m_new)
    p = jnp.exp(z - m_new)
    s_ref[0, 0] = s_ref[0, 0] * corr + jnp.sum(p)
    pv = lax.dot_general(p, combined, (((0,), (0,)), ((), ())),
                         preferred_element_type=jnp.float32)
    v_ref[...] = v_ref[...] * corr + pv
    m_ref[0, 0] = m_new

    @pl.when(i == nblk - 1)
    def _():
        code = v_ref[...] / s_ref[0, 0]
        y = lax.dot_general(code, wd_ref[...], (((1,), (0,)), ((), ())),
                            preferred_element_type=jnp.float32) + b_ref[...]
        out_ref[...] = 1.0 / (1.0 + jnp.exp(-y))


def _tc_attn(ctx3, W, a_row, W_dense, b_row, n):
    nblk = n // _BN
    small = lambda i: (0, 0)
    return pl.pallas_call(
        functools.partial(_attn_body, nblk=nblk),
        grid=(nblk,),
        in_specs=[
            pl.BlockSpec((1, _BN, D), lambda i: (0, i, 0)),
            pl.BlockSpec((1, _BN, D), lambda i: (1, i, 0)),
            pl.BlockSpec((1, _BN, D), lambda i: (2, i, 0)),
            pl.BlockSpec((D, 3 * D), small),
            pl.BlockSpec((1, D), small),
            pl.BlockSpec((D, D), small),
            pl.BlockSpec((1, D), small),
        ],
        out_specs=pl.BlockSpec((1, D), small),
        out_shape=jax.ShapeDtypeStruct((1, D), jnp.float32),
        scratch_shapes=[
            pltpu.SMEM((1, 1), jnp.float32),
            pltpu.SMEM((1, 1), jnp.float32),
            pltpu.VMEM((1, D), jnp.float32),
        ],
    )(ctx3, ctx3, ctx3, W, a_row, W_dense, b_row)


def kernel(inputs, value_vocab, path_vocab, W, attention_vector, W_dense, b_dense):
    n = inputs.shape[0]
    idx_flat = inputs.astype(jnp.int32).T.reshape(3 * n)
    ctx = _sc_gather(value_vocab, path_vocab, idx_flat, n)
    ctx3 = ctx.reshape(3, n, D)
    return _tc_attn(ctx3, W, attention_vector.reshape(1, D), W_dense,
                    b_dense.reshape(1, D), n)
